# baseline (speedup 1.0000x reference)
import functools

import jax
import jax.numpy as jnp
from jax import lax
from jax.experimental import pallas as pl
from jax.experimental.pallas import tpu as pltpu
from jax.experimental.pallas import tpu_sc as plsc

_NC = 2
_NS = 16
_NW = _NC * _NS
_F = 16

_CK = 400
_GB = 80
_CK2 = 2000

_MESH = plsc.VectorSubcoreMesh(core_axis_name="c", subcore_axis_name="s")


def _proj_body(x_ref, w_ref, p_ref, q_ref):
    pq = jnp.dot(x_ref[...], w_ref[...], preferred_element_type=jnp.float32)
    p_ref[...] = pq[:, :_F]
    q_ref[...] = pq[:, _F:]


def _project(x, wcat, block_rows=1000):
    n, d = x.shape
    return pl.pallas_call(
        _proj_body,
        grid=(n // block_rows,),
        in_specs=[pl.BlockSpec((block_rows, d), lambda i: (i, 0)),
                  pl.BlockSpec((d, 2 * _F), lambda i: (0, 0))],
        out_specs=[pl.BlockSpec((block_rows, _F), lambda i: (i, 0)),
                   pl.BlockSpec((block_rows, _F), lambda i: (i, 0))],
        out_shape=[jax.ShapeDtypeStruct((n, _F), jnp.float32),
                   jax.ShapeDtypeStruct((n, _F), jnp.float32)],
    )(x, wcat)


def _merge_proj_body(part_ref, w_ref, p_ref, q_ref):
    h = jnp.max(part_ref[...], axis=0)
    h = jnp.maximum(h, 0.0)
    pq = jnp.dot(h, w_ref[...], preferred_element_type=jnp.float32)
    p_ref[...] = pq[:, :_F]
    q_ref[...] = pq[:, _F:]


def _merge_project(parts, wcat, block_rows=1000):
    _, n, f = parts.shape
    return pl.pallas_call(
        _merge_proj_body,
        grid=(n // block_rows,),
        in_specs=[pl.BlockSpec((_NW, block_rows, f), lambda i: (0, i, 0)),
                  pl.BlockSpec((f, 2 * _F), lambda i: (0, 0))],
        out_specs=[pl.BlockSpec((block_rows, _F), lambda i: (i, 0)),
                   pl.BlockSpec((block_rows, _F), lambda i: (i, 0))],
        out_shape=[jax.ShapeDtypeStruct((n, _F), jnp.float32),
                   jax.ShapeDtypeStruct((n, _F), jnp.float32)],
    )(parts, wcat)


def _final_body(part_ref, o_ref):
    a = jnp.max(part_ref[...], axis=0)
    a = jnp.where(a == -jnp.inf, 0.0, a)
    z = a[:, :7]
    z = z - jnp.max(z, axis=1, keepdims=True)
    o_ref[...] = z - jnp.log(jnp.sum(jnp.exp(z), axis=1, keepdims=True))


def _final(parts, block_rows=1000):
    _, n, f = parts.shape
    return pl.pallas_call(
        _final_body,
        grid=(n // block_rows,),
        in_specs=[pl.BlockSpec((_NW, block_rows, f), lambda i: (0, i, 0))],
        out_specs=pl.BlockSpec((block_rows, 7), lambda i: (i, 0)),
        out_shape=jax.ShapeDtypeStruct((n, 7), jnp.float32),
    )(parts)


def _tree_sum(terms):
    t = list(terms)
    while len(t) > 1:
        nxt = [t[i] + t[i + 1] for i in range(0, len(t) - 1, 2)]
        if len(t) % 2:
            nxt.append(t[-1])
        t = nxt
    return t[0]


def _msg_body(p_hbm, q_hbm, src_hbm, dst_hbm, wb_hbm, ba_hbm, bb_hbm, m_hbm,
              dstv, srcv, av, bv, mv, wbv, babbv, sem):
    wid = lax.axis_index("s") * _NC + lax.axis_index("c")
    ew = m_hbm.shape[0] // _NW
    pltpu.sync_copy(wb_hbm, wbv)
    pltpu.sync_copy(ba_hbm, babbv.at[0])
    pltpu.sync_copy(bb_hbm, babbv.at[1])
    ba = babbv[0, :]
    bb = babbv[1, :]
    wrows = [wbv[k, :] for k in range(_F)]

    def chunk(c, carry):
        base = wid * ew + c * _CK
        pltpu.sync_copy(dst_hbm.at[pl.ds(base, _CK)], dstv)
        pltpu.sync_copy(src_hbm.at[pl.ds(base, _CK)], srcv)
        cps = []
        for s in range(_CK // _GB):
            sl = pl.ds(s * _GB, _GB)
            cps.append(pltpu.async_copy(p_hbm.at[dstv.at[sl]], av.at[sl], sem))
            cps.append(pltpu.async_copy(q_hbm.at[srcv.at[sl]], bv.at[sl], sem))
        for cp in cps:
            cp.wait()

        @plsc.parallel_loop(0, _CK, unroll=4)
        def edge(e):
            u = jnp.maximum(av[e, :] + bv[e, :] + ba, 0.0)
            terms = [u.at[lax.full((_F,), k, jnp.int32)].get(
                         mode="promise_in_bounds", unique_indices=False,
                         indices_are_sorted=True) * wrows[k]
                     for k in range(_F)]
            mv[e, :] = bb + _tree_sum(terms)
        pltpu.sync_copy(mv, m_hbm.at[pl.ds(base, _CK)])
        return carry

    lax.fori_loop(0, ew // _CK, chunk, 0)


def _msg(p_tab, q_tab, src, dst, wb, ba, bb):
    e = src.shape[0]
    return pl.kernel(
        _msg_body,
        out_type=jax.ShapeDtypeStruct((e, _F), jnp.float32),
        mesh=_MESH,
        scratch_types=[
            pltpu.VMEM((_CK,), jnp.int32),
            pltpu.VMEM((_CK,), jnp.int32),
            pltpu.VMEM((_CK, _F), jnp.float32),
            pltpu.VMEM((_CK, _F), jnp.float32),
            pltpu.VMEM((_CK, _F), jnp.float32),
            pltpu.VMEM((_F, _F), jnp.float32),
            pltpu.VMEM((2, _F), jnp.float32),
            pltpu.SemaphoreType.DMA,
        ],
        compiler_params=pltpu.CompilerParams(use_tc_tiling_on_sc=False),
    )(p_tab, q_tab, src, dst, wb, ba, bb)


def _segmax_body(m_hbm, dst_hbm, out_hbm, accv, mv, dstv):
    wid = lax.axis_index("s") * _NC + lax.axis_index("c")
    ew = m_hbm.shape[0] // _NW
    n = out_hbm.shape[1]
    half = n // 2
    lanes = lax.iota(jnp.int32, _F)
    for hh in range(2):
        lo = hh * half

        def initr(i, c):
            accv[i, :] = jnp.full((_F,), -jnp.inf, jnp.float32)
            return c

        lax.fori_loop(0, half + 8, initr, 0, unroll=16)

        def chunk(c, carry):
            base = pl.multiple_of(wid * ew + c * _CK2, 8)
            pltpu.sync_copy(m_hbm.at[pl.ds(base, _CK2)], mv)
            pltpu.sync_copy(dst_hbm.at[pl.ds(base, _CK2)], dstv)

            @plsc.parallel_loop(0, _CK2, unroll=8)
            def edge(e):
                d = plsc.load_gather(dstv, [jnp.full((_F,), e, jnp.int32)]) - lo
                ok = (d >= 0) & (d < half)
                rv = jnp.where(ok, d, half)
                cur = plsc.load_gather(accv, [rv, lanes])
                plsc.store_scatter(accv, [rv, lanes],
                                   jnp.maximum(cur, mv[e, :]))

            def grp(g, c2):
                e0 = g * _F
                eids = e0 + lanes
                d = plsc.load_gather(dstv, [eids]) - lo
                ok = (d >= 0) & (d < half)
                rv = jnp.where(ok, d, half)
                bad = jnp.zeros((_F,), jnp.bool_)
                for f in range(_F):
                    fv = jnp.full((_F,), f, jnp.int32)
                    af = plsc.load_gather(accv, [rv, fv])
                    vf = plsc.load_gather(mv, [eids, fv])
                    bad = bad | (af < vf)
                bad = bad & ok

                @pl.when(jnp.any(bad))
                def _fix():
                    def fix(e, c3):
                        d2 = plsc.load_gather(
                            dstv, [jnp.full((_F,), e, jnp.int32)]) - lo
                        ok2 = (d2 >= 0) & (d2 < half)
                        rv2 = jnp.where(ok2, d2, half)
                        cur = plsc.load_gather(accv, [rv2, lanes])
                        plsc.store_scatter(accv, [rv2, lanes],
                                           jnp.maximum(cur, mv[e, :]))
                        return c3
                    lax.fori_loop(e0, e0 + _F, fix, 0)
                return c2

            lax.fori_loop(0, _CK2 // _F, grp, 0)
            return carry

        lax.fori_loop(0, ew // _CK2, chunk, 0)
        pltpu.sync_copy(accv.at[pl.ds(0, half)], out_hbm.at[wid].at[pl.ds(lo, half)])


def _segmax(m, dst, n):
    e = m.shape[0]
    return pl.kernel(
        _segmax_body,
        out_type=jax.ShapeDtypeStruct((_NW, n, _F), jnp.float32),
        mesh=_MESH,
        scratch_types=[
            pltpu.VMEM((n // 2 + 8, _F), jnp.float32),
            pltpu.VMEM((_CK2, _F), jnp.float32),
            pltpu.VMEM((_CK2,), jnp.int32),
        ],
        compiler_params=pltpu.CompilerParams(use_tc_tiling_on_sc=False,
                                             needs_layout_passes=False),
    )(m, dst)


def kernel(x, edge_index, W1a, b1a, W1b, b1b, W2a, b2a, W2b, b2b):
    n, d = x.shape
    src = edge_index[0]
    dst = edge_index[1]

    w1cat = jnp.concatenate([W1a[:d] - W1a[d:], W1a[d:]], axis=1)
    p1, q1 = _project(x, w1cat)
    m1 = _msg(p1, q1, src, dst, W1b, b1a, b1b)
    parts1 = _segmax(m1, dst, n)

    h = _F
    w2cat = (jnp.zeros((h, 2 * _F), jnp.float32)
             .at[:, :7].set(W2a[:h] - W2a[h:])
             .at[:, _F:_F + 7].set(W2a[h:]))
    p2, q2 = _merge_project(parts1, w2cat)

    w2b_pad = jnp.zeros((_F, _F), jnp.float32).at[:7, :7].set(W2b)
    b2a_pad = jnp.zeros((_F,), jnp.float32).at[:7].set(b2a)
    b2b_pad = jnp.zeros((_F,), jnp.float32).at[:7].set(b2b)
    m2 = _msg(p2, q2, src, dst, w2b_pad, b2a_pad, b2b_pad)
    parts2 = _segmax(m2, dst, n)
    return _final(parts2)


# speedup vs baseline: 1.2558x; 1.2558x over previous
import functools

import jax
import jax.numpy as jnp
from jax import lax
from jax.experimental import pallas as pl
from jax.experimental.pallas import tpu as pltpu
from jax.experimental.pallas import tpu_sc as plsc

_NC = 2
_NS = 16
_NW = _NC * _NS
_F = 16

_CK = 400
_GB = 80
_CK2 = 2000

_MESH = plsc.VectorSubcoreMesh(core_axis_name="c", subcore_axis_name="s")


def _proj_body(x_ref, w_ref, p_ref, q_ref):
    pq = jnp.dot(x_ref[...], w_ref[...], preferred_element_type=jnp.float32)
    p_ref[...] = pq[:, :_F]
    q_ref[...] = pq[:, _F:]


def _project(x, wcat, block_rows=1000):
    n, d = x.shape
    return pl.pallas_call(
        _proj_body,
        grid=(n // block_rows,),
        in_specs=[pl.BlockSpec((block_rows, d), lambda i: (i, 0)),
                  pl.BlockSpec((d, 2 * _F), lambda i: (0, 0))],
        out_specs=[pl.BlockSpec((block_rows, _F), lambda i: (i, 0)),
                   pl.BlockSpec((block_rows, _F), lambda i: (i, 0))],
        out_shape=[jax.ShapeDtypeStruct((n, _F), jnp.float32),
                   jax.ShapeDtypeStruct((n, _F), jnp.float32)],
    )(x, wcat)


def _merge_proj_body(part_ref, w_ref, p_ref, q_ref):
    h = jnp.max(part_ref[...], axis=0)
    h = jnp.maximum(h, 0.0)
    pq = jnp.dot(h, w_ref[...], preferred_element_type=jnp.float32)
    p_ref[...] = pq[:, :_F]
    q_ref[...] = pq[:, _F:]


def _merge_project(parts, wcat, block_rows=1000):
    _, n, f = parts.shape
    return pl.pallas_call(
        _merge_proj_body,
        grid=(n // block_rows,),
        in_specs=[pl.BlockSpec((_NC, block_rows, f), lambda i: (0, i, 0)),
                  pl.BlockSpec((f, 2 * _F), lambda i: (0, 0))],
        out_specs=[pl.BlockSpec((block_rows, _F), lambda i: (i, 0)),
                   pl.BlockSpec((block_rows, _F), lambda i: (i, 0))],
        out_shape=[jax.ShapeDtypeStruct((n, _F), jnp.float32),
                   jax.ShapeDtypeStruct((n, _F), jnp.float32)],
    )(parts, wcat)


def _final_body(part_ref, o_ref):
    a = jnp.max(part_ref[...], axis=0)
    a = jnp.where(a == -jnp.inf, 0.0, a)
    z = a[:, :7]
    z = z - jnp.max(z, axis=1, keepdims=True)
    o_ref[...] = z - jnp.log(jnp.sum(jnp.exp(z), axis=1, keepdims=True))


def _final(parts, block_rows=1000):
    _, n, f = parts.shape
    return pl.pallas_call(
        _final_body,
        grid=(n // block_rows,),
        in_specs=[pl.BlockSpec((_NC, block_rows, f), lambda i: (0, i, 0))],
        out_specs=pl.BlockSpec((block_rows, 7), lambda i: (i, 0)),
        out_shape=jax.ShapeDtypeStruct((n, 7), jnp.float32),
    )(parts)


def _tree_sum(terms):
    t = list(terms)
    while len(t) > 1:
        nxt = [t[i] + t[i + 1] for i in range(0, len(t) - 1, 2)]
        if len(t) % 2:
            nxt.append(t[-1])
        t = nxt
    return t[0]


def _msg_body(p_hbm, q_hbm, src_hbm, dst_hbm, wb_hbm, ba_hbm, bb_hbm, m_hbm,
              dstv, srcv, av, bv, mv, wbv, babbv, sem):
    wid = lax.axis_index("s") * _NC + lax.axis_index("c")
    ew = m_hbm.shape[0] // _NW
    pltpu.sync_copy(wb_hbm, wbv)
    pltpu.sync_copy(ba_hbm, babbv.at[0])
    pltpu.sync_copy(bb_hbm, babbv.at[1])
    ba = babbv[0, :]
    bb = babbv[1, :]
    wrows = [wbv[k, :] for k in range(_F)]

    def chunk(c, carry):
        base = wid * ew + c * _CK
        pltpu.sync_copy(dst_hbm.at[pl.ds(base, _CK)], dstv)
        pltpu.sync_copy(src_hbm.at[pl.ds(base, _CK)], srcv)
        cps = []
        for s in range(_CK // _GB):
            sl = pl.ds(s * _GB, _GB)
            cps.append(pltpu.async_copy(p_hbm.at[dstv.at[sl]], av.at[sl], sem))
            cps.append(pltpu.async_copy(q_hbm.at[srcv.at[sl]], bv.at[sl], sem))
        for cp in cps:
            cp.wait()

        @plsc.parallel_loop(0, _CK)
        def edge(e):
            u = jnp.maximum(av[e, :] + bv[e, :] + ba, 0.0)
            terms = [u.at[lax.full((_F,), k, jnp.int32)].get(
                         mode="promise_in_bounds", unique_indices=False,
                         indices_are_sorted=True) * wrows[k]
                     for k in range(_F)]
            mv[e, :] = bb + _tree_sum(terms)
        pltpu.sync_copy(mv, m_hbm.at[pl.ds(base, _CK)])
        return carry

    lax.fori_loop(0, ew // _CK, chunk, 0)


def _msg(p_tab, q_tab, src, dst, wb, ba, bb):
    e = src.shape[0]
    return pl.kernel(
        _msg_body,
        out_type=jax.ShapeDtypeStruct((e, _F), jnp.float32),
        mesh=_MESH,
        scratch_types=[
            pltpu.VMEM((_CK,), jnp.int32),
            pltpu.VMEM((_CK,), jnp.int32),
            pltpu.VMEM((_CK, _F), jnp.float32),
            pltpu.VMEM((_CK, _F), jnp.float32),
            pltpu.VMEM((_CK, _F), jnp.float32),
            pltpu.VMEM((_F, _F), jnp.float32),
            pltpu.VMEM((2, _F), jnp.float32),
            pltpu.SemaphoreType.DMA,
        ],
        compiler_params=pltpu.CompilerParams(use_tc_tiling_on_sc=False),
    )(p_tab, q_tab, src, dst, wb, ba, bb)


def _segmax_body(m_hbm, dst_hbm, out_hbm, part_hbm, accv, mv, dstv, mrgv, tmpv):
    sid = lax.axis_index("s")
    cid = lax.axis_index("c")
    wid = sid * _NC + cid
    ew = m_hbm.shape[0] // _NW
    n = out_hbm.shape[1]
    half = n // 2
    rows_main = (half // _NS) // 8 * 8
    rows_rest = half - _NS * rows_main
    lanes = lax.iota(jnp.int32, _F)

    def merge_rows(lo, r0, nrows):
        pc = part_hbm.at[cid]
        pltpu.sync_copy(pc.at[0].at[pl.ds(lo + r0, nrows)],
                        mrgv.at[pl.ds(0, nrows)])
        for s in range(1, _NS):
            pltpu.sync_copy(pc.at[s].at[pl.ds(lo + r0, nrows)],
                            tmpv.at[pl.ds(0, nrows)])

            def mrow(i, c3):
                mrgv[i, :] = jnp.maximum(mrgv[i, :], tmpv[i, :])
                return c3

            lax.fori_loop(0, nrows, mrow, 0, unroll=8)
        pltpu.sync_copy(mrgv.at[pl.ds(0, nrows)],
                        out_hbm.at[cid].at[pl.ds(lo + r0, nrows)])
    for hh in range(2):
        lo = hh * half

        def initr(i, c):
            accv[i, :] = jnp.full((_F,), -jnp.inf, jnp.float32)
            return c

        lax.fori_loop(0, half + 8, initr, 0, unroll=16)

        def chunk(c, carry):
            base = pl.multiple_of(wid * ew + c * _CK2, 8)
            pltpu.sync_copy(m_hbm.at[pl.ds(base, _CK2)], mv)
            pltpu.sync_copy(dst_hbm.at[pl.ds(base, _CK2)], dstv)

            @plsc.parallel_loop(0, _CK2)
            def edge(e):
                d = plsc.load_gather(dstv, [jnp.full((_F,), e, jnp.int32)]) - lo
                ok = (d >= 0) & (d < half)
                rv = jnp.where(ok, d, half)
                cur = plsc.load_gather(accv, [rv, lanes])
                plsc.store_scatter(accv, [rv, lanes],
                                   jnp.maximum(cur, mv[e, :]))

            def grp(g, c2):
                e0 = g * _F
                eids = e0 + lanes
                d = plsc.load_gather(dstv, [eids]) - lo
                ok = (d >= 0) & (d < half)
                rv = jnp.where(ok, d, half)
                bad = jnp.zeros((_F,), jnp.bool_)
                for f in range(_F):
                    fv = jnp.full((_F,), f, jnp.int32)
                    af = plsc.load_gather(accv, [rv, fv])
                    vf = plsc.load_gather(mv, [eids, fv])
                    bad = bad | (af < vf)
                bad = bad & ok

                @pl.when(jnp.any(bad))
                def _fix():
                    def fix(e, c3):
                        d2 = plsc.load_gather(
                            dstv, [jnp.full((_F,), e, jnp.int32)]) - lo
                        ok2 = (d2 >= 0) & (d2 < half)
                        rv2 = jnp.where(ok2, d2, half)
                        cur = plsc.load_gather(accv, [rv2, lanes])
                        plsc.store_scatter(accv, [rv2, lanes],
                                           jnp.maximum(cur, mv[e, :]))
                        return c3
                    lax.fori_loop(e0, e0 + _F, fix, 0)
                return c2

            lax.fori_loop(0, _CK2 // _F, grp, 0)
            return carry

        lax.fori_loop(0, ew // _CK2, chunk, 0)
        pltpu.sync_copy(accv.at[pl.ds(0, half)],
                        part_hbm.at[cid].at[sid].at[pl.ds(lo, half)])
        plsc.subcore_barrier()
        r0 = pl.multiple_of(sid * rows_main, 8)
        merge_rows(lo, r0, rows_main)
        if rows_rest:
            @pl.when(sid == _NS - 1)
            def _rest():
                merge_rows(lo, _NS * rows_main, rows_rest)


def _segmax(m, dst, n):
    rows_buf = (n // 2 // _NS) // 8 * 8 + 8
    out, _parts = pl.kernel(
        _segmax_body,
        out_type=[jax.ShapeDtypeStruct((_NC, n, _F), jnp.float32),
                  jax.ShapeDtypeStruct((_NC, _NS, n, _F), jnp.float32)],
        mesh=_MESH,
        scratch_types=[
            pltpu.VMEM((n // 2 + 8, _F), jnp.float32),
            pltpu.VMEM((_CK2, _F), jnp.float32),
            pltpu.VMEM((_CK2,), jnp.int32),
            pltpu.VMEM((rows_buf, _F), jnp.float32),
            pltpu.VMEM((rows_buf, _F), jnp.float32),
        ],
        compiler_params=pltpu.CompilerParams(use_tc_tiling_on_sc=False,
                                             needs_layout_passes=False),
    )(m, dst)
    return out


def kernel(x, edge_index, W1a, b1a, W1b, b1b, W2a, b2a, W2b, b2b):
    n, d = x.shape
    src = edge_index[0]
    dst = edge_index[1]

    w1cat = jnp.concatenate([W1a[:d] - W1a[d:], W1a[d:]], axis=1)
    p1, q1 = _project(x, w1cat)
    m1 = _msg(p1, q1, src, dst, W1b, b1a, b1b)
    parts1 = _segmax(m1, dst, n)

    h = _F
    w2cat = (jnp.zeros((h, 2 * _F), jnp.float32)
             .at[:, :7].set(W2a[:h] - W2a[h:])
             .at[:, _F:_F + 7].set(W2a[h:]))
    p2, q2 = _merge_project(parts1, w2cat)

    w2b_pad = jnp.zeros((_F, _F), jnp.float32).at[:7, :7].set(W2b)
    b2a_pad = jnp.zeros((_F,), jnp.float32).at[:7].set(b2a)
    b2b_pad = jnp.zeros((_F,), jnp.float32).at[:7].set(b2b)
    m2 = _msg(p2, q2, src, dst, w2b_pad, b2a_pad, b2b_pad)
    parts2 = _segmax(m2, dst, n)
    return _final(parts2)


# speedup vs baseline: 1.2558x; 1.0000x over previous
import functools

import jax
import jax.numpy as jnp
from jax import lax
from jax.experimental import pallas as pl
from jax.experimental.pallas import tpu as pltpu
from jax.experimental.pallas import tpu_sc as plsc

_NC = 2
_NS = 16
_NW = _NC * _NS
_F = 16

_CK = 400
_GB = 80
_CK2 = 2000

_MESH = plsc.VectorSubcoreMesh(core_axis_name="c", subcore_axis_name="s")


def _proj_body(x_ref, w_ref, p_ref, q_ref):
    x = x_ref[...]
    w = w_ref[...]
    d = x.shape[1]
    wbot = w[d:]
    p_ref[...] = jnp.dot(x, w[:d] - wbot, preferred_element_type=jnp.float32)
    q_ref[...] = jnp.dot(x, wbot, preferred_element_type=jnp.float32)


def _project(x, wa, block_rows=1000):
    n, d = x.shape
    return pl.pallas_call(
        _proj_body,
        grid=(n // block_rows,),
        in_specs=[pl.BlockSpec((block_rows, d), lambda i: (i, 0)),
                  pl.BlockSpec((2 * d, _F), lambda i: (0, 0))],
        out_specs=[pl.BlockSpec((block_rows, _F), lambda i: (i, 0)),
                   pl.BlockSpec((block_rows, _F), lambda i: (i, 0))],
        out_shape=[jax.ShapeDtypeStruct((n, _F), jnp.float32),
                   jax.ShapeDtypeStruct((n, _F), jnp.float32)],
    )(x, wa)


def _merge_proj_body(part_ref, w_ref, p_ref, q_ref):
    h = jnp.max(part_ref[...], axis=0)
    h = jnp.maximum(h, 0.0)
    w = w_ref[...]
    wbot = w[_F:]
    pad = jnp.zeros((h.shape[0], _F - 7), jnp.float32)
    ph = jnp.dot(h, w[:_F] - wbot, preferred_element_type=jnp.float32)
    qh = jnp.dot(h, wbot, preferred_element_type=jnp.float32)
    p_ref[...] = jnp.concatenate([ph, pad], axis=1)
    q_ref[...] = jnp.concatenate([qh, pad], axis=1)


def _merge_project(parts, w2a, block_rows=1000):
    _, n, f = parts.shape
    return pl.pallas_call(
        _merge_proj_body,
        grid=(n // block_rows,),
        in_specs=[pl.BlockSpec((_NC, block_rows, f), lambda i: (0, i, 0)),
                  pl.BlockSpec((2 * _F, 7), lambda i: (0, 0))],
        out_specs=[pl.BlockSpec((block_rows, _F), lambda i: (i, 0)),
                   pl.BlockSpec((block_rows, _F), lambda i: (i, 0))],
        out_shape=[jax.ShapeDtypeStruct((n, _F), jnp.float32),
                   jax.ShapeDtypeStruct((n, _F), jnp.float32)],
    )(parts, w2a)


def _final_body(part_ref, o_ref):
    a = jnp.max(part_ref[...], axis=0)
    a = jnp.where(a == -jnp.inf, 0.0, a)
    z = a[:, :7]
    z = z - jnp.max(z, axis=1, keepdims=True)
    o_ref[...] = z - jnp.log(jnp.sum(jnp.exp(z), axis=1, keepdims=True))


def _final(parts, block_rows=1000):
    _, n, f = parts.shape
    return pl.pallas_call(
        _final_body,
        grid=(n // block_rows,),
        in_specs=[pl.BlockSpec((_NC, block_rows, f), lambda i: (0, i, 0))],
        out_specs=pl.BlockSpec((block_rows, 7), lambda i: (i, 0)),
        out_shape=jax.ShapeDtypeStruct((n, 7), jnp.float32),
    )(parts)


def _tree_sum(terms):
    t = list(terms)
    while len(t) > 1:
        nxt = [t[i] + t[i + 1] for i in range(0, len(t) - 1, 2)]
        if len(t) % 2:
            nxt.append(t[-1])
        t = nxt
    return t[0]


def _msg_body(p_hbm, q_hbm, src_hbm, dst_hbm, wb_hbm, ba_hbm, bb_hbm, m_hbm,
              dstv, srcv, av, bv, mv, wbv, babbv, sem):
    wid = lax.axis_index("s") * _NC + lax.axis_index("c")
    ew = m_hbm.shape[0] // _NW
    pltpu.sync_copy(wb_hbm, wbv)
    pltpu.sync_copy(ba_hbm, babbv.at[0])
    pltpu.sync_copy(bb_hbm, babbv.at[1])
    ba = babbv[0, :]
    bb = babbv[1, :]
    wrows = [wbv[k, :] for k in range(_F)]

    def chunk(c, carry):
        base = wid * ew + c * _CK
        pltpu.sync_copy(dst_hbm.at[pl.ds(base, _CK)], dstv)
        pltpu.sync_copy(src_hbm.at[pl.ds(base, _CK)], srcv)
        cps = []
        for s in range(_CK // _GB):
            sl = pl.ds(s * _GB, _GB)
            cps.append(pltpu.async_copy(p_hbm.at[dstv.at[sl]], av.at[sl], sem))
            cps.append(pltpu.async_copy(q_hbm.at[srcv.at[sl]], bv.at[sl], sem))
        for cp in cps:
            cp.wait()

        @plsc.parallel_loop(0, _CK)
        def edge(e):
            u = jnp.maximum(av[e, :] + bv[e, :] + ba, 0.0)

            def bc(k):
                return u.at[lax.full((_F,), k, jnp.int32)].get(
                    mode="promise_in_bounds", unique_indices=False,
                    indices_are_sorted=True)

            accs = [bb + bc(0) * wrows[0]] + [bc(k) * wrows[k]
                                              for k in range(1, 4)]
            for k in range(4, _F):
                accs[k % 4] = accs[k % 4] + bc(k) * wrows[k]
            mv[e, :] = (accs[0] + accs[1]) + (accs[2] + accs[3])
        pltpu.sync_copy(mv, m_hbm.at[pl.ds(base, _CK)])
        return carry

    lax.fori_loop(0, ew // _CK, chunk, 0)


def _msg(p_tab, q_tab, src, dst, wb, ba, bb):
    e = src.shape[0]
    return pl.kernel(
        _msg_body,
        out_type=jax.ShapeDtypeStruct((e, _F), jnp.float32),
        mesh=_MESH,
        scratch_types=[
            pltpu.VMEM((_CK,), jnp.int32),
            pltpu.VMEM((_CK,), jnp.int32),
            pltpu.VMEM((_CK, _F), jnp.float32),
            pltpu.VMEM((_CK, _F), jnp.float32),
            pltpu.VMEM((_CK, _F), jnp.float32),
            pltpu.VMEM((_F, _F), jnp.float32),
            pltpu.VMEM((2, _F), jnp.float32),
            pltpu.SemaphoreType.DMA,
        ],
        compiler_params=pltpu.CompilerParams(use_tc_tiling_on_sc=False),
    )(p_tab, q_tab, src, dst, wb, ba, bb)


def _segmax_body(m_hbm, dst_hbm, out_hbm, part_hbm, accv, mv, dstv, mrgv, tmpv):
    sid = lax.axis_index("s")
    cid = lax.axis_index("c")
    wid = sid * _NC + cid
    ew = m_hbm.shape[0] // _NW
    n = out_hbm.shape[1]
    half = n // 2
    rows_main = (half // _NS) // 8 * 8
    rows_rest = half - _NS * rows_main
    lanes = lax.iota(jnp.int32, _F)

    def merge_rows(lo, r0, nrows):
        pc = part_hbm.at[cid]
        pltpu.sync_copy(pc.at[0].at[pl.ds(lo + r0, nrows)],
                        mrgv.at[pl.ds(0, nrows)])
        for s in range(1, _NS):
            pltpu.sync_copy(pc.at[s].at[pl.ds(lo + r0, nrows)],
                            tmpv.at[pl.ds(0, nrows)])

            def mrow(i, c3):
                mrgv[i, :] = jnp.maximum(mrgv[i, :], tmpv[i, :])
                return c3

            lax.fori_loop(0, nrows, mrow, 0, unroll=8)
        pltpu.sync_copy(mrgv.at[pl.ds(0, nrows)],
                        out_hbm.at[cid].at[pl.ds(lo + r0, nrows)])
    for hh in range(2):
        lo = hh * half

        def initr(i, c):
            accv[i, :] = jnp.full((_F,), -jnp.inf, jnp.float32)
            return c

        lax.fori_loop(0, half + 8, initr, 0, unroll=16)

        def chunk(c, carry):
            base = pl.multiple_of(wid * ew + c * _CK2, 8)
            pltpu.sync_copy(m_hbm.at[pl.ds(base, _CK2)], mv)
            pltpu.sync_copy(dst_hbm.at[pl.ds(base, _CK2)], dstv)

            @plsc.parallel_loop(0, _CK2)
            def edge(e):
                d = plsc.load_gather(dstv, [jnp.full((_F,), e, jnp.int32)]) - lo
                ok = (d >= 0) & (d < half)
                rv = jnp.where(ok, d, half)
                cur = plsc.load_gather(accv, [rv, lanes])
                plsc.store_scatter(accv, [rv, lanes],
                                   jnp.maximum(cur, mv[e, :]))

            def grp(g, c2):
                e0 = g * _F
                eids = e0 + lanes
                d = plsc.load_gather(dstv, [eids]) - lo
                ok = (d >= 0) & (d < half)
                rv = jnp.where(ok, d, half)
                bad = jnp.zeros((_F,), jnp.bool_)
                for f in range(_F):
                    fv = jnp.full((_F,), f, jnp.int32)
                    af = plsc.load_gather(accv, [rv, fv])
                    vf = plsc.load_gather(mv, [eids, fv])
                    bad = bad | (af < vf)
                bad = bad & ok

                @pl.when(jnp.any(bad))
                def _fix():
                    def fix(e, c3):
                        d2 = plsc.load_gather(
                            dstv, [jnp.full((_F,), e, jnp.int32)]) - lo
                        ok2 = (d2 >= 0) & (d2 < half)
                        rv2 = jnp.where(ok2, d2, half)
                        cur = plsc.load_gather(accv, [rv2, lanes])
                        plsc.store_scatter(accv, [rv2, lanes],
                                           jnp.maximum(cur, mv[e, :]))
                        return c3
                    lax.fori_loop(e0, e0 + _F, fix, 0)
                return c2

            lax.fori_loop(0, _CK2 // _F, grp, 0)
            return carry

        lax.fori_loop(0, ew // _CK2, chunk, 0)
        pltpu.sync_copy(accv.at[pl.ds(0, half)],
                        part_hbm.at[cid].at[sid].at[pl.ds(lo, half)])
        plsc.subcore_barrier()
        r0 = pl.multiple_of(sid * rows_main, 8)
        merge_rows(lo, r0, rows_main)
        if rows_rest:
            @pl.when(sid == _NS - 1)
            def _rest():
                merge_rows(lo, _NS * rows_main, rows_rest)


def _segmax(m, dst, n):
    rows_buf = (n // 2 // _NS) // 8 * 8 + 8
    out, _parts = pl.kernel(
        _segmax_body,
        out_type=[jax.ShapeDtypeStruct((_NC, n, _F), jnp.float32),
                  jax.ShapeDtypeStruct((_NC, _NS, n, _F), jnp.float32)],
        mesh=_MESH,
        scratch_types=[
            pltpu.VMEM((n // 2 + 8, _F), jnp.float32),
            pltpu.VMEM((_CK2, _F), jnp.float32),
            pltpu.VMEM((_CK2,), jnp.int32),
            pltpu.VMEM((rows_buf, _F), jnp.float32),
            pltpu.VMEM((rows_buf, _F), jnp.float32),
        ],
        compiler_params=pltpu.CompilerParams(use_tc_tiling_on_sc=False,
                                             needs_layout_passes=False),
    )(m, dst)
    return out


def kernel(x, edge_index, W1a, b1a, W1b, b1b, W2a, b2a, W2b, b2b):
    n, d = x.shape
    src = edge_index[0]
    dst = edge_index[1]

    p1, q1 = _project(x, W1a)
    m1 = _msg(p1, q1, src, dst, W1b, b1a, b1b)
    parts1 = _segmax(m1, dst, n)
    p2, q2 = _merge_project(parts1, W2a)

    w2b_pad = jnp.zeros((_F, _F), jnp.float32).at[:7, :7].set(W2b)
    b2a_pad = jnp.zeros((_F,), jnp.float32).at[:7].set(b2a)
    b2b_pad = jnp.zeros((_F,), jnp.float32).at[:7].set(b2b)
    m2 = _msg(p2, q2, src, dst, w2b_pad, b2a_pad, b2b_pad)
    parts2 = _segmax(m2, dst, n)
    return _final(parts2)


# speedup vs baseline: 1.4173x; 1.1286x over previous
import functools

import jax
import jax.numpy as jnp
from jax import lax
from jax.experimental import pallas as pl
from jax.experimental.pallas import tpu as pltpu
from jax.experimental.pallas import tpu_sc as plsc

_NC = 2
_NS = 16
_NW = _NC * _NS
_F = 16

_CK = 400
_GB = 80
_CK2 = 2000

_MESH = plsc.VectorSubcoreMesh(core_axis_name="c", subcore_axis_name="s")


def _proj_body(x_ref, w_ref, p_ref, q_ref):
    x = x_ref[...]
    w = w_ref[...]
    d = x.shape[1]
    wbot = w[d:]
    p_ref[...] = jnp.dot(x, w[:d] - wbot, preferred_element_type=jnp.float32)
    q_ref[...] = jnp.dot(x, wbot, preferred_element_type=jnp.float32)


def _project(x, wa, block_rows=1000):
    n, d = x.shape
    return pl.pallas_call(
        _proj_body,
        grid=(n // block_rows,),
        in_specs=[pl.BlockSpec((block_rows, d), lambda i: (i, 0)),
                  pl.BlockSpec((2 * d, _F), lambda i: (0, 0))],
        out_specs=[pl.BlockSpec((block_rows, _F), lambda i: (i, 0)),
                   pl.BlockSpec((block_rows, _F), lambda i: (i, 0))],
        out_shape=[jax.ShapeDtypeStruct((n, _F), jnp.float32),
                   jax.ShapeDtypeStruct((n, _F), jnp.float32)],
    )(x, wa)


def _merge_proj_body(part_ref, w_ref, p_ref, q_ref):
    h = jnp.max(part_ref[...], axis=0)
    h = jnp.maximum(h, 0.0)
    w = w_ref[...]
    wbot = w[_F:]
    pad = jnp.zeros((h.shape[0], _F - 7), jnp.float32)
    ph = jnp.dot(h, w[:_F] - wbot, preferred_element_type=jnp.float32)
    qh = jnp.dot(h, wbot, preferred_element_type=jnp.float32)
    p_ref[...] = jnp.concatenate([ph, pad], axis=1)
    q_ref[...] = jnp.concatenate([qh, pad], axis=1)


def _merge_project(parts, w2a, block_rows=1000):
    _, n, f = parts.shape
    return pl.pallas_call(
        _merge_proj_body,
        grid=(n // block_rows,),
        in_specs=[pl.BlockSpec((_NC, block_rows, f), lambda i: (0, i, 0)),
                  pl.BlockSpec((2 * _F, 7), lambda i: (0, 0))],
        out_specs=[pl.BlockSpec((block_rows, _F), lambda i: (i, 0)),
                   pl.BlockSpec((block_rows, _F), lambda i: (i, 0))],
        out_shape=[jax.ShapeDtypeStruct((n, _F), jnp.float32),
                   jax.ShapeDtypeStruct((n, _F), jnp.float32)],
    )(parts, w2a)


def _final_body(part_ref, o_ref):
    a = jnp.max(part_ref[...], axis=0)
    a = jnp.where(a == -jnp.inf, 0.0, a)
    z = a[:, :7]
    z = z - jnp.max(z, axis=1, keepdims=True)
    o_ref[...] = z - jnp.log(jnp.sum(jnp.exp(z), axis=1, keepdims=True))


def _final(parts, block_rows=1000):
    _, n, f = parts.shape
    return pl.pallas_call(
        _final_body,
        grid=(n // block_rows,),
        in_specs=[pl.BlockSpec((_NC, block_rows, f), lambda i: (0, i, 0))],
        out_specs=pl.BlockSpec((block_rows, 7), lambda i: (i, 0)),
        out_shape=jax.ShapeDtypeStruct((n, 7), jnp.float32),
    )(parts)


def _tree_sum(terms):
    t = list(terms)
    while len(t) > 1:
        nxt = [t[i] + t[i + 1] for i in range(0, len(t) - 1, 2)]
        if len(t) % 2:
            nxt.append(t[-1])
        t = nxt
    return t[0]


def _msg_body(p_hbm, q_hbm, src_hbm, dst_hbm, wb_hbm, ba_hbm, bb_hbm, m_hbm,
              dstv, srcv, av, bv, mv, wbv, babbv, sem):
    wid = lax.axis_index("s") * _NC + lax.axis_index("c")
    ew = m_hbm.shape[0] // _NW
    pltpu.sync_copy(wb_hbm, wbv)
    pltpu.sync_copy(ba_hbm, babbv.at[0])
    pltpu.sync_copy(bb_hbm, babbv.at[1])
    ba = babbv[0, :]
    bb = babbv[1, :]
    wrows = [wbv[k, :] for k in range(_F)]

    def chunk(c, carry):
        base = wid * ew + c * _CK
        pltpu.sync_copy(dst_hbm.at[pl.ds(base, _CK)], dstv)
        pltpu.sync_copy(src_hbm.at[pl.ds(base, _CK)], srcv)
        cps = []
        for s in range(_CK // _GB):
            sl = pl.ds(s * _GB, _GB)
            cps.append(pltpu.async_copy(p_hbm.at[dstv.at[sl]], av.at[sl], sem))
            cps.append(pltpu.async_copy(q_hbm.at[srcv.at[sl]], bv.at[sl], sem))
        for cp in cps:
            cp.wait()

        @plsc.parallel_loop(0, _CK)
        def edge(e):
            u = jnp.maximum(av[e, :] + bv[e, :] + ba, 0.0)

            def bc(k):
                return u.at[lax.full((_F,), k, jnp.int32)].get(
                    mode="promise_in_bounds", unique_indices=False,
                    indices_are_sorted=True)

            accs = [bb + bc(0) * wrows[0]] + [bc(k) * wrows[k]
                                              for k in range(1, 4)]
            for k in range(4, _F):
                accs[k % 4] = accs[k % 4] + bc(k) * wrows[k]
            mv[e, :] = (accs[0] + accs[1]) + (accs[2] + accs[3])
        pltpu.sync_copy(mv, m_hbm.at[pl.ds(base, _CK)])
        return carry

    lax.fori_loop(0, ew // _CK, chunk, 0)


def _msg(p_tab, q_tab, src, dst, wb, ba, bb):
    e = src.shape[0]
    return pl.kernel(
        _msg_body,
        out_type=jax.ShapeDtypeStruct((e, _F), jnp.float32),
        mesh=_MESH,
        scratch_types=[
            pltpu.VMEM((_CK,), jnp.int32),
            pltpu.VMEM((_CK,), jnp.int32),
            pltpu.VMEM((_CK, _F), jnp.float32),
            pltpu.VMEM((_CK, _F), jnp.float32),
            pltpu.VMEM((_CK, _F), jnp.float32),
            pltpu.VMEM((_F, _F), jnp.float32),
            pltpu.VMEM((2, _F), jnp.float32),
            pltpu.SemaphoreType.DMA,
        ],
        compiler_params=pltpu.CompilerParams(use_tc_tiling_on_sc=False),
    )(p_tab, q_tab, src, dst, wb, ba, bb)


def _segmax_body(m_hbm, dst_hbm, out_hbm, part_hbm, accv, mv, dstv, mrgv, sem):
    sid = lax.axis_index("s")
    cid = lax.axis_index("c")
    wid = sid * _NC + cid
    ew = m_hbm.shape[0] // _NW
    n = out_hbm.shape[1]
    half = n // 2
    rows_main = (half // _NS) // 8 * 8
    rows_rest = half - _NS * rows_main
    lanes = lax.iota(jnp.int32, _F)

    def merge_rows(lo, r0, nrows):
        pc = part_hbm.at[cid]
        cps = [pltpu.async_copy(pc.at[s].at[pl.ds(lo + r0, nrows)],
                                accv.at[pl.ds(s * nrows, nrows)], sem)
               for s in range(_NS)]
        for cp in cps:
            cp.wait()

        @plsc.parallel_loop(0, nrows)
        def mrow(i):
            vs = [accv[s * nrows + i, :] for s in range(_NS)]
            while len(vs) > 1:
                nxt = [jnp.maximum(vs[j], vs[j + 1])
                       for j in range(0, len(vs) - 1, 2)]
                if len(vs) % 2:
                    nxt.append(vs[-1])
                vs = nxt
            mrgv[i, :] = vs[0]

        pltpu.sync_copy(mrgv.at[pl.ds(0, nrows)],
                        out_hbm.at[cid].at[pl.ds(lo + r0, nrows)])
    for hh in range(2):
        lo = hh * half

        def initr(i, c):
            accv[i, :] = jnp.full((_F,), -jnp.inf, jnp.float32)
            return c

        lax.fori_loop(0, half + 8, initr, 0, unroll=16)

        def chunk(c, carry):
            base = pl.multiple_of(wid * ew + c * _CK2, 8)
            pltpu.sync_copy(m_hbm.at[pl.ds(base, _CK2)], mv)
            pltpu.sync_copy(dst_hbm.at[pl.ds(base, _CK2)], dstv)

            @plsc.parallel_loop(0, _CK2)
            def edge(e):
                d = plsc.load_gather(dstv, [jnp.full((_F,), e, jnp.int32)]) - lo
                ok = (d >= 0) & (d < half)
                rv = jnp.where(ok, d, half)
                cur = plsc.load_gather(accv, [rv, lanes])
                plsc.store_scatter(accv, [rv, lanes],
                                   jnp.maximum(cur, mv[e, :]))

            def grp(g, c2):
                e0 = g * _F
                eids = e0 + lanes
                d = plsc.load_gather(dstv, [eids]) - lo
                ok = (d >= 0) & (d < half)
                rv = jnp.where(ok, d, half)
                bad = jnp.zeros((_F,), jnp.bool_)
                for f in range(_F):
                    fv = jnp.full((_F,), f, jnp.int32)
                    af = plsc.load_gather(accv, [rv, fv])
                    vf = plsc.load_gather(mv, [eids, fv])
                    bad = bad | (af < vf)
                bad = bad & ok

                @pl.when(jnp.any(bad))
                def _fix():
                    def fix(e, c3):
                        d2 = plsc.load_gather(
                            dstv, [jnp.full((_F,), e, jnp.int32)]) - lo
                        ok2 = (d2 >= 0) & (d2 < half)
                        rv2 = jnp.where(ok2, d2, half)
                        cur = plsc.load_gather(accv, [rv2, lanes])
                        plsc.store_scatter(accv, [rv2, lanes],
                                           jnp.maximum(cur, mv[e, :]))
                        return c3
                    lax.fori_loop(e0, e0 + _F, fix, 0)
                return c2

            lax.fori_loop(0, _CK2 // _F, grp, 0)
            return carry

        lax.fori_loop(0, ew // _CK2, chunk, 0)
        pltpu.sync_copy(accv.at[pl.ds(0, half)],
                        part_hbm.at[cid].at[sid].at[pl.ds(lo, half)])
        plsc.subcore_barrier()
        r0 = pl.multiple_of(sid * rows_main, 8)
        merge_rows(lo, r0, rows_main)
        if rows_rest:
            @pl.when(sid == _NS - 1)
            def _rest():
                merge_rows(lo, _NS * rows_main, rows_rest)


def _segmax(m, dst, n):
    rows_buf = (n // 2 // _NS) // 8 * 8 + 8
    out, _parts = pl.kernel(
        _segmax_body,
        out_type=[jax.ShapeDtypeStruct((_NC, n, _F), jnp.float32),
                  jax.ShapeDtypeStruct((_NC, _NS, n, _F), jnp.float32)],
        mesh=_MESH,
        scratch_types=[
            pltpu.VMEM((n // 2 + 8, _F), jnp.float32),
            pltpu.VMEM((_CK2, _F), jnp.float32),
            pltpu.VMEM((_CK2,), jnp.int32),
            pltpu.VMEM((rows_buf, _F), jnp.float32),
            pltpu.SemaphoreType.DMA,
        ],
        compiler_params=pltpu.CompilerParams(use_tc_tiling_on_sc=False,
                                             needs_layout_passes=False),
    )(m, dst)
    return out


def kernel(x, edge_index, W1a, b1a, W1b, b1b, W2a, b2a, W2b, b2b):
    n, d = x.shape
    src = edge_index[0]
    dst = edge_index[1]

    p1, q1 = _project(x, W1a)
    m1 = _msg(p1, q1, src, dst, W1b, b1a, b1b)
    parts1 = _segmax(m1, dst, n)
    p2, q2 = _merge_project(parts1, W2a)

    w2b_pad = jnp.zeros((_F, _F), jnp.float32).at[:7, :7].set(W2b)
    b2a_pad = jnp.zeros((_F,), jnp.float32).at[:7].set(b2a)
    b2b_pad = jnp.zeros((_F,), jnp.float32).at[:7].set(b2b)
    m2 = _msg(p2, q2, src, dst, w2b_pad, b2a_pad, b2b_pad)
    parts2 = _segmax(m2, dst, n)
    return _final(parts2)
